# baseline (speedup 1.0000x reference)
import jax
import jax.numpy as jnp
from jax import lax
from jax.experimental import pallas as pl
from jax.experimental.pallas import tpu as pltpu
from jax.experimental.pallas import tpu_sc as plsc

B = 16384
F = 26
V = 100000
D = 16

NC = 2
NS = 16
NW = NC * NS
BPW = B // NW
FSPLIT = 8
K = 8


def _make_sc_body(nf):
    nrow = BPW * nf // 128

    def body(idx_hbm, wflat_hbm, out_hbm, idx_v, vals_v, lin_v, sem):
        wid = lax.axis_index("s") * NC + lax.axis_index("c")
        pltpu.sync_copy(idx_hbm.at[wid], idx_v)

        for t in range(K):
            pltpu.async_copy(wflat_hbm.at[idx_v.at[t]], vals_v.at[t], sem)

        def chunk(i, carry):
            base = i * K
            for t in range(K):
                j = base + K + t
                pltpu.async_copy(wflat_hbm.at[idx_v.at[j]], vals_v.at[j], sem)
            for t in range(K):
                j = base + t
                pltpu.make_async_copy(
                    wflat_hbm.at[idx_v.at[j]], vals_v.at[j], sem
                ).wait()
            return carry

        lax.fori_loop(0, nrow // K - 1, chunk, 0)
        for t in range(K):
            j = nrow - K + t
            pltpu.make_async_copy(
                wflat_hbm.at[idx_v.at[j]], vals_v.at[j], sem
            ).wait()

        for c in range(BPW // 16):
            r0 = c // 8
            col = (c % 8) * 16
            acc = vals_v[r0, pl.ds(col, 16)]
            for f in range(1, nf):
                acc = acc + vals_v[f * 4 + r0, pl.ds(col, 16)]
            lin_v[pl.ds(c * 16, 16)] = acc
        pltpu.sync_copy(lin_v, out_hbm.at[pl.ds(wid * BPW, BPW)])

    return body, nrow


def _sc_linear(idx, wflat, nf):
    body, nrow = _make_sc_body(nf)
    mesh = plsc.VectorSubcoreMesh(core_axis_name="c", subcore_axis_name="s")
    return pl.kernel(
        body,
        out_type=jax.ShapeDtypeStruct((B,), jnp.float32),
        mesh=mesh,
        scratch_types=[
            pltpu.VMEM((nrow, 128), jnp.int32),
            pltpu.VMEM((nrow, 128), jnp.float32),
            pltpu.VMEM((BPW,), jnp.float32),
            pltpu.SemaphoreType.DMA,
        ],
    )(idx, wflat)


def _worker_chunks(xt, f0, nf):
    off = (jnp.arange(nf, dtype=jnp.int32) * V)[:, None]
    fidx = xt[f0 : f0 + nf] + off
    return fidx.reshape(nf, NW, BPW).transpose(1, 0, 2).reshape(NW, -1, 128)


def _tc_inter_body(fe_ref, out_ref):
    acc = fe_ref[0]
    acc2 = acc * acc
    for f in range(1, F):
        v = fe_ref[f]
        acc = acc + v
        acc2 = acc2 + v * v
    out_ref[...] = (acc * acc - acc2) * 0.5


def _tc_interaction(fe_t):
    bt = 4096
    return pl.pallas_call(
        _tc_inter_body,
        grid=(B // bt,),
        in_specs=[pl.BlockSpec((F, D, bt), lambda i: (0, 0, i))],
        out_specs=pl.BlockSpec((D, bt), lambda i: (0, i)),
        out_shape=jax.ShapeDtypeStruct((D, B), jnp.float32),
    )(fe_t)


def kernel(x, feature_emb, w_linear, bias):
    fe_t = feature_emb.transpose(1, 2, 0)
    xt = x.T.astype(jnp.int32)
    wf_a = w_linear[:FSPLIT].reshape(-1)
    wf_b = w_linear[FSPLIT:].reshape(-1)
    idx_a = _worker_chunks(xt, 0, FSPLIT)
    idx_b = _worker_chunks(xt, FSPLIT, F - FSPLIT)
    lin_a = _sc_linear(idx_a, wf_a, FSPLIT)
    lin_b = _sc_linear(idx_b, wf_b, F - FSPLIT)
    inter_t = _tc_interaction(fe_t)
    return (inter_t + (lin_a + lin_b + bias[0])[None, :]).T


# speedup vs baseline: 1.1505x; 1.1505x over previous
import jax
import jax.numpy as jnp
from jax import lax
from jax.experimental import pallas as pl
from jax.experimental.pallas import tpu as pltpu
from jax.experimental.pallas import tpu_sc as plsc

B = 16384
F = 26
V = 100000
D = 16

NC = 2
NS = 16
NW = NC * NS
BPW = B // NW
NROW = BPW * F // 128
P = 24


def _sc_lin_body(idx_hbm, wflat_hbm, out_hbm, idx_v, vals_v, lin_v, sem):
    wid = lax.axis_index("s") * NC + lax.axis_index("c")
    pltpu.sync_copy(idx_hbm.at[wid], idx_v)

    for j in range(P):
        pltpu.async_copy(wflat_hbm.at[idx_v.at[j]], vals_v.at[j], sem)

    def step(j, carry):
        pltpu.async_copy(wflat_hbm.at[idx_v.at[j + P]], vals_v.at[j + P], sem)
        pltpu.make_async_copy(
            wflat_hbm.at[idx_v.at[j]], vals_v.at[j], sem
        ).wait()
        return carry

    lax.fori_loop(0, NROW - P, step, 0)
    for j in range(NROW - P, NROW):
        pltpu.make_async_copy(wflat_hbm.at[idx_v.at[j]], vals_v.at[j], sem).wait()

    for c in range(BPW // 16):
        r0 = c // 8
        col = (c % 8) * 16
        acc = vals_v[r0, pl.ds(col, 16)]
        for f in range(1, F):
            acc = acc + vals_v[f * 4 + r0, pl.ds(col, 16)]
        lin_v[pl.ds(c * 16, 16)] = acc
    pltpu.sync_copy(lin_v, out_hbm.at[pl.ds(wid * BPW, BPW)])


def _sc_linear(idx, wflat):
    mesh = plsc.VectorSubcoreMesh(core_axis_name="c", subcore_axis_name="s")
    return pl.kernel(
        _sc_lin_body,
        out_type=jax.ShapeDtypeStruct((B,), jnp.float32),
        mesh=mesh,
        scratch_types=[
            pltpu.VMEM((NROW, 128), jnp.int32),
            pltpu.VMEM((NROW, 128), jnp.float32),
            pltpu.VMEM((BPW,), jnp.float32),
            pltpu.SemaphoreType.DMA,
        ],
    )(idx, wflat)


def _tc_inter_body(fe_ref, out_ref):
    acc = fe_ref[0]
    acc2 = acc * acc
    for f in range(1, F):
        v = fe_ref[f]
        acc = acc + v
        acc2 = acc2 + v * v
    out_ref[...] = (acc * acc - acc2) * 0.5


def _tc_interaction(fe_t):
    bt = 4096
    return pl.pallas_call(
        _tc_inter_body,
        grid=(B // bt,),
        in_specs=[pl.BlockSpec((F, D, bt), lambda i: (0, 0, i))],
        out_specs=pl.BlockSpec((D, bt), lambda i: (0, i)),
        out_shape=jax.ShapeDtypeStruct((D, B), jnp.float32),
    )(fe_t)


def kernel(x, feature_emb, w_linear, bias):
    fe_t = feature_emb.transpose(1, 2, 0)
    wflat = w_linear.reshape(F * V)
    fidx = x.T.astype(jnp.int32) + (jnp.arange(F, dtype=jnp.int32) * V)[:, None]
    fidx = fidx.reshape(F, NW, BPW).transpose(1, 0, 2).reshape(NW, NROW, 128)
    lin = _sc_linear(fidx, wflat)
    inter_t = _tc_interaction(fe_t)
    return (inter_t + (lin + bias[0])[None, :]).T


# speedup vs baseline: 1.3902x; 1.2084x over previous
import functools

import jax
import jax.numpy as jnp
from jax import lax
from jax.experimental import pallas as pl
from jax.experimental.pallas import tpu as pltpu
from jax.experimental.pallas import tpu_sc as plsc

B = 16384
F = 26
V = 100000
D = 16

NC = 2
NS = 16
TABW = 100224
CHUNK = 2048


def _sc_lin_body(off_hbm, wflat_hbm, out_hbm, tab_v, idx_v, out_v, sem):
    cid = lax.axis_index("c")
    sid = lax.axis_index("s")
    wid = jnp.minimum(sid * NC + cid, F - 1)

    start = pl.multiple_of((wid * V // 128) * 128, 128)
    a = pltpu.async_copy(wflat_hbm.at[pl.ds(start, TABW)], tab_v, sem)
    b = pltpu.async_copy(off_hbm.at[pl.ds(wid * B, B)], idx_v, sem)
    a.wait()
    b.wait()

    for c in range(B // CHUNK):
        for i in range(CHUNK // 16):
            iv = idx_v[pl.ds(c * CHUNK + i * 16, 16)]
            out_v[pl.ds(i * 16, 16)] = plsc.load_gather(tab_v, [iv])
        pltpu.sync_copy(
            out_v, out_hbm.at[pl.ds(wid * B + c * CHUNK, CHUNK)]
        )


def _sc_linear(off, wflat):
    mesh = plsc.VectorSubcoreMesh(core_axis_name="c", subcore_axis_name="s")
    return pl.kernel(
        _sc_lin_body,
        out_type=jax.ShapeDtypeStruct((F * B,), jnp.float32),
        mesh=mesh,
        compiler_params=pltpu.CompilerParams(needs_layout_passes=False),
        scratch_types=[
            pltpu.VMEM((TABW,), jnp.float32),
            pltpu.VMEM((B,), jnp.int32),
            pltpu.VMEM((CHUNK,), jnp.float32),
            pltpu.SemaphoreType.DMA,
        ],
    )(off, wflat)


def _tc_inter_body(fe_ref, out_ref):
    acc = fe_ref[0]
    acc2 = acc * acc
    for f in range(1, F):
        v = fe_ref[f]
        acc = acc + v
        acc2 = acc2 + v * v
    out_ref[...] = (acc * acc - acc2) * 0.5


def _tc_interaction(fe_t):
    bt = 4096
    return pl.pallas_call(
        _tc_inter_body,
        grid=(B // bt,),
        in_specs=[pl.BlockSpec((F, D, bt), lambda i: (0, 0, i))],
        out_specs=pl.BlockSpec((D, bt), lambda i: (0, i)),
        out_shape=jax.ShapeDtypeStruct((D, B), jnp.float32),
    )(fe_t)


def kernel(x, feature_emb, w_linear, bias):
    fe_t = feature_emb.transpose(1, 2, 0)
    wflat = w_linear.reshape(F * V)
    delta = (jnp.arange(F, dtype=jnp.int32) * V) % 128
    off = (x.T.astype(jnp.int32) + delta[:, None]).reshape(F * B)
    vals = _sc_linear(off, wflat)
    inter_t = _tc_interaction(fe_t)
    lin = functools.reduce(
        lambda a, b: a + b, [vals[f * B : (f + 1) * B] for f in range(F)]
    )
    return (inter_t + (lin + bias[0])[None, :]).T


# speedup vs baseline: 1.4422x; 1.0375x over previous
import functools

import jax
import jax.numpy as jnp
from jax import lax
from jax.experimental import pallas as pl
from jax.experimental.pallas import tpu as pltpu
from jax.experimental.pallas import tpu_sc as plsc

B = 16384
F = 26
V = 100000
D = 16

NC = 2
NS = 16
TABW = 100224
CHUNK = 2048


def _sc_lin_body(off_hbm, wflat_hbm, out_hbm, tab_v, idx_v, out_v, sem, osem):
    cid = lax.axis_index("c")
    sid = lax.axis_index("s")
    wid = sid * NC + cid

    @pl.when(wid < F)
    def _():
        start = pl.multiple_of((wid * V // 128) * 128, 128)
        a = pltpu.async_copy(wflat_hbm.at[pl.ds(start, TABW)], tab_v, sem)
        b = pltpu.async_copy(off_hbm.at[pl.ds(wid * B, B)], idx_v, sem)
        a.wait()
        b.wait()

        pend = []
        for c in range(B // CHUNK):
            buf = c % 2
            if len(pend) == 2:
                pend.pop(0).wait()
            for i in range(CHUNK // 16):
                iv = idx_v[pl.ds(c * CHUNK + i * 16, 16)]
                out_v[buf, pl.ds(i * 16, 16)] = plsc.load_gather(tab_v, [iv])
            pend.append(
                pltpu.async_copy(
                    out_v.at[buf],
                    out_hbm.at[pl.ds(wid * B + c * CHUNK, CHUNK)],
                    osem,
                )
            )
        for p in pend:
            p.wait()


def _sc_linear(off, wflat):
    mesh = plsc.VectorSubcoreMesh(core_axis_name="c", subcore_axis_name="s")
    return pl.kernel(
        _sc_lin_body,
        out_type=jax.ShapeDtypeStruct((F * B,), jnp.float32),
        mesh=mesh,
        compiler_params=pltpu.CompilerParams(needs_layout_passes=False),
        scratch_types=[
            pltpu.VMEM((TABW,), jnp.float32),
            pltpu.VMEM((B,), jnp.int32),
            pltpu.VMEM((2, CHUNK), jnp.float32),
            pltpu.SemaphoreType.DMA,
            pltpu.SemaphoreType.DMA,
        ],
    )(off, wflat)


def _tc_inter_body(fe_ref, out_ref):
    acc = fe_ref[0]
    acc2 = acc * acc
    for f in range(1, F):
        v = fe_ref[f]
        acc = acc + v
        acc2 = acc2 + v * v
    out_ref[...] = (acc * acc - acc2) * 0.5


def _tc_interaction(fe_t):
    bt = 8192
    return pl.pallas_call(
        _tc_inter_body,
        grid=(B // bt,),
        in_specs=[pl.BlockSpec((F, D, bt), lambda i: (0, 0, i))],
        out_specs=pl.BlockSpec((D, bt), lambda i: (0, i)),
        out_shape=jax.ShapeDtypeStruct((D, B), jnp.float32),
    )(fe_t)


def kernel(x, feature_emb, w_linear, bias):
    fe_t = feature_emb.transpose(1, 2, 0)
    wflat = w_linear.reshape(F * V)
    delta = (jnp.arange(F, dtype=jnp.int32) * V) % 128
    off = (x.T.astype(jnp.int32) + delta[:, None]).reshape(F * B)
    vals = _sc_linear(off, wflat)
    inter_t = _tc_interaction(fe_t)
    lin = functools.reduce(
        lambda a, b: a + b, [vals[f * B : (f + 1) * B] for f in range(F)]
    )
    return (inter_t + (lin + bias[0])[None, :]).T
